# baseline (speedup 1.0000x reference)
import jax
import jax.numpy as jnp
from jax import lax
from jax.experimental import pallas as pl
from jax.experimental.pallas import tpu as pltpu
from jax.experimental.pallas import tpu_sc as plsc

_B = 8
_G = 2048
_NS = 64
_NTILES = 32
_N = _B * _G * _G
_SLAB = _N // _NTILES
_SPT = (_B * _NS) // _NTILES


def _sc_body(y_hbm, x_hbm, yi_hbm, xi_hbm, f_hbm, o_hbm,
             yi_v, xi_v, eidx_v, xv_v, f_v, vals_v, sem_big, sem_idx):
    wid = lax.axis_index("s") * 2 + lax.axis_index("c")
    e0 = wid * _SLAB
    big = pltpu.async_copy(y_hbm.at[pl.ds(e0, _SLAB)], o_hbm.at[pl.ds(e0, _SLAB)],
                           sem_big)
    s0 = wid * _SPT
    i0 = lax.rem(s0, _NS)
    b = s0 // _NS
    pltpu.sync_copy(yi_hbm.at[pl.ds(i0, _SPT)], yi_v)
    pltpu.sync_copy(xi_hbm.at[pl.ds(i0, _SPT)], xi_v)
    pltpu.sync_copy(x_hbm.at[pl.ds(s0, _SPT)], xv_v)
    pltpu.sync_copy(f_hbm, f_v)
    eidx_v[...] = (yi_v[...] + b * _G) * _G + xi_v[...]
    pltpu.async_copy(y_hbm.at[eidx_v], vals_v, sem_idx).wait()
    vals_v[...] = vals_v[...] + f_v[...] * xv_v[...]
    big.wait()
    pltpu.async_copy(vals_v, o_hbm.at[eidx_v], sem_idx).wait()


def kernel(Y, X, y_idx, x_idx, f):
    Yf = Y.reshape(_N)
    Xf = X.reshape(_B * _NS)
    f_arr = jnp.full((16,), f, jnp.float32)
    mesh = plsc.VectorSubcoreMesh(core_axis_name="c", subcore_axis_name="s")
    out = pl.kernel(
        _sc_body,
        out_type=jax.ShapeDtypeStruct((_N,), jnp.float32),
        mesh=mesh,
        scratch_types=[
            pltpu.VMEM((_SPT,), jnp.int32),
            pltpu.VMEM((_SPT,), jnp.int32),
            pltpu.VMEM((_SPT,), jnp.int32),
            pltpu.VMEM((_SPT,), jnp.float32),
            pltpu.VMEM((16,), jnp.float32),
            pltpu.VMEM((_SPT,), jnp.float32),
            pltpu.SemaphoreType.DMA,
            pltpu.SemaphoreType.DMA,
        ],
    )(Yf, Xf, y_idx, x_idx, f_arr)
    return out.reshape(_B, _G, _G)


# speedup vs baseline: 50.5371x; 50.5371x over previous
import jax
import jax.numpy as jnp
from jax import lax
from jax.experimental import pallas as pl
from jax.experimental.pallas import tpu as pltpu

_B = 8
_G = 2048
_NS = 64
_R = 1024


def _body(y_ref, x_ref, yi_ref, xi_ref, f_ref, o_ref):
    j = pl.program_id(1)
    o_ref[...] = y_ref[...]
    r0 = j * _R
    fval = f_ref[0, 0]
    col = lax.broadcasted_iota(jnp.int32, (1, _G), 1)

    def step(s, carry):
        y = yi_ref[s]
        x = xi_ref[s]
        row = y - r0

        @pl.when((row >= 0) & (row < _R))
        def _():
            v = fval * x_ref[0, 0, s]
            o_ref[0, pl.ds(row, 1), :] += jnp.where(col == x, v, 0.0)

        return carry

    lax.fori_loop(0, _NS, step, 0)


def kernel(Y, X, y_idx, x_idx, f):
    f_arr = jnp.asarray(f, jnp.float32).reshape(1, 1)
    grid = (_B, _G // _R)
    return pl.pallas_call(
        _body,
        grid=grid,
        in_specs=[
            pl.BlockSpec((1, _R, _G), lambda b, j: (b, j, 0)),
            pl.BlockSpec((1, 1, _NS), lambda b, j: (b, 0, 0), memory_space=pltpu.SMEM),
            pl.BlockSpec((_NS,), lambda b, j: (0,), memory_space=pltpu.SMEM),
            pl.BlockSpec((_NS,), lambda b, j: (0,), memory_space=pltpu.SMEM),
            pl.BlockSpec((1, 1), lambda b, j: (0, 0), memory_space=pltpu.SMEM),
        ],
        out_specs=pl.BlockSpec((1, _R, _G), lambda b, j: (b, j, 0)),
        out_shape=jax.ShapeDtypeStruct((_B, _G, _G), jnp.float32),
        compiler_params=pltpu.CompilerParams(
            dimension_semantics=("arbitrary", "arbitrary"),
        ),
    )(Y, X.reshape(_B, 1, _NS), y_idx, x_idx, f_arr)
